# baseline (speedup 1.0000x reference)
import functools

import jax
import jax.numpy as jnp
from jax import lax
from jax.experimental import pallas as pl
from jax.experimental.pallas import tpu as pltpu
from jax.experimental.pallas import tpu_sc as plsc

VOCAB = 1000000
EMBED = 16
BATCH = 16384
HIST = 200

NC = 2
NS = 16
L = 16
NW = NC * NS
ROWS_PER_W = BATCH // NW
GROUPS_PER_W = ROWS_PER_W // L
IDX_PER_GROUP = HIST * L

PROJ_BLK = 262144

P_BNC = 32000
P_SUB = 2 * P_BNC
P_TAIL = VOCAB - 15 * P_SUB - P_BNC


def _proj_body(t_ref, w_ref, o_ref):
    mm = jax.lax.dot_general(w_ref[...], t_ref[...],
                             (((1,), (0,)), ((), ())),
                             preferred_element_type=jnp.float32)
    o_ref[...] = mm[0, :]


def _project(table_t, wcol):
    grid = (VOCAB + PROJ_BLK - 1) // PROJ_BLK
    return pl.pallas_call(
        _proj_body,
        grid=(grid,),
        in_specs=[
            pl.BlockSpec((EMBED, PROJ_BLK), lambda i: (0, i)),
            pl.BlockSpec((8, EMBED), lambda i: (0, 0)),
        ],
        out_specs=pl.BlockSpec((PROJ_BLK,), lambda i: (i,)),
        out_shape=jax.ShapeDtypeStruct((VOCAB,), jnp.float32),
    )(table_t, wcol)


def _sigmoid_round(logit):
    neg = logit < 0.0
    t = jnp.exp(jnp.where(neg, logit, -logit))
    sig = jnp.where(neg, t / (1.0 + t), 1.0 / (1.0 + t))
    y = sig * 10000.0
    r = (y + 8388608.0) - 8388608.0
    return r / 10000.0


def _pool_body(x1d_hbm, p_hbm, b_hbm, out_hbm, idx_v, idx_w, vals_v, vals_w,
               out_v, b_v, p_sh, p_bnc, semg0, semg1, semi0, semi1):
    cid = lax.axis_index("c")
    sid = lax.axis_index("s")
    wid = sid * NC + cid

    def stage_chunk(off, size):
        pltpu.sync_copy(p_hbm.at[pl.ds(off, size)],
                        p_bnc.at[pl.ds(0, size)])
        pltpu.sync_copy(p_bnc.at[pl.ds(0, size)],
                        p_sh.at[pl.ds(off, size)])

    stage_chunk(sid * P_SUB, P_BNC)

    @pl.when(sid < NS - 1)
    def _():
        stage_chunk(sid * P_SUB + P_BNC, P_BNC)

    @pl.when(sid == NS - 1)
    def _():
        stage_chunk((NS - 1) * P_SUB + P_BNC, P_TAIL)

    pltpu.sync_copy(b_hbm, b_v)
    bvec = b_v[...]
    base_lanes = lax.iota(jnp.int32, L) * HIST
    plsc.subcore_barrier()

    idx_bufs = (idx_v, idx_w)
    val_bufs = (vals_v, vals_w)
    gsems = (semg0, semg1)
    isems = (semi0, semi1)
    g_base = wid * GROUPS_PER_W

    def idx_src(g):
        return x1d_hbm.at[pl.ds((g_base + g) * IDX_PER_GROUP, IDX_PER_GROUP)]

    def reduce_store(g, vals):
        accs = [jnp.zeros((L,), jnp.float32) for _ in range(4)]
        for i in range(HIST):
            accs[i % 4] = accs[i % 4] + plsc.load_gather(
                vals, [base_lanes + i])
        logit = (accs[0] + accs[1]) + (accs[2] + accs[3]) + bvec
        out_v[pl.ds(g * L, L)] = _sigmoid_round(logit)

    pltpu.sync_copy(idx_src(0), idx_bufs[0])
    pltpu.async_copy(p_sh.at[idx_bufs[0]], val_bufs[0], gsems[0])
    pltpu.async_copy(idx_src(1), idx_bufs[1], isems[1])

    def pair_body(i, carry):
        for b in (0, 1):
            g = 2 * i + b
            nb = 1 - b

            @pl.when(g + 1 < GROUPS_PER_W)
            def _():
                pltpu.make_async_copy(idx_src(g + 1), idx_bufs[nb],
                                      isems[nb]).wait()
                pltpu.async_copy(p_sh.at[idx_bufs[nb]], val_bufs[nb],
                                 gsems[nb])

            pltpu.make_async_copy(p_sh.at[idx_bufs[b]], val_bufs[b],
                                  gsems[b]).wait()

            @pl.when(g + 2 < GROUPS_PER_W)
            def _():
                pltpu.async_copy(idx_src(g + 2), idx_bufs[b], isems[b])

            reduce_store(g, val_bufs[b])
        return carry

    lax.fori_loop(0, GROUPS_PER_W // 2, pair_body, 0)
    pltpu.sync_copy(out_v, out_hbm.at[pl.ds(wid * ROWS_PER_W, ROWS_PER_W)])


@functools.cache
def _build_pool_kernel():
    return pl.kernel(
        _pool_body,
        out_type=jax.ShapeDtypeStruct((BATCH,), jnp.float32),
        mesh=plsc.VectorSubcoreMesh(core_axis_name="c", subcore_axis_name="s",
                                    num_cores=NC, num_subcores=NS),
        scratch_types=[
            pltpu.VMEM((IDX_PER_GROUP,), jnp.int32),
            pltpu.VMEM((IDX_PER_GROUP,), jnp.int32),
            pltpu.VMEM((IDX_PER_GROUP,), jnp.float32),
            pltpu.VMEM((IDX_PER_GROUP,), jnp.float32),
            pltpu.VMEM((ROWS_PER_W,), jnp.float32),
            pltpu.VMEM((L,), jnp.float32),
            pltpu.VMEM_SHARED((VOCAB,), jnp.float32),
            pltpu.VMEM((P_BNC,), jnp.float32),
            pltpu.SemaphoreType.DMA,
            pltpu.SemaphoreType.DMA,
            pltpu.SemaphoreType.DMA,
            pltpu.SemaphoreType.DMA,
        ],
        compiler_params=pltpu.CompilerParams(needs_layout_passes=False),
    )


def kernel(x, table, W, b):
    w8 = jnp.broadcast_to(W.reshape(1, EMBED) * (1.0 / HIST), (8, EMBED))
    p = _project(table.T, w8)
    x1d = x.reshape(BATCH * HIST)
    b16 = jnp.broadcast_to(b.astype(jnp.float32), (L,))
    out = _build_pool_kernel()(x1d, p, b16)
    return out.reshape(BATCH, 1)


# speedup vs baseline: 1.0062x; 1.0062x over previous
import functools

import jax
import jax.numpy as jnp
from jax import lax
from jax.experimental import pallas as pl
from jax.experimental.pallas import tpu as pltpu
from jax.experimental.pallas import tpu_sc as plsc

VOCAB = 1000000
EMBED = 16
BATCH = 16384
HIST = 200

NC = 2
NS = 16
L = 16
NW = NC * NS
ROWS_PER_W = BATCH // NW
GROUPS_PER_W = ROWS_PER_W // L
IDX_PER_GROUP = HIST * L

PROJ_BLK = 131072

P_BNC = 32000
P_SUB = 2 * P_BNC
P_TAIL = VOCAB - 15 * P_SUB - P_BNC


def _proj_body(t_ref, w_ref, o_ref):
    mm = jax.lax.dot_general(w_ref[...], t_ref[...],
                             (((1,), (0,)), ((), ())),
                             preferred_element_type=jnp.float32)
    o_ref[...] = mm[0, :]


def _project(table_t, wcol):
    grid = (VOCAB + PROJ_BLK - 1) // PROJ_BLK
    return pl.pallas_call(
        _proj_body,
        grid=(grid,),
        in_specs=[
            pl.BlockSpec((EMBED, PROJ_BLK), lambda i: (0, i)),
            pl.BlockSpec((8, EMBED), lambda i: (0, 0)),
        ],
        out_specs=pl.BlockSpec((PROJ_BLK,), lambda i: (i,)),
        out_shape=jax.ShapeDtypeStruct((VOCAB,), jnp.float32),
    )(table_t, wcol)


def _sigmoid_round(logit):
    neg = logit < 0.0
    t = jnp.exp(jnp.where(neg, logit, -logit))
    sig = jnp.where(neg, t / (1.0 + t), 1.0 / (1.0 + t))
    y = sig * 10000.0
    r = (y + 8388608.0) - 8388608.0
    return r / 10000.0


def _pool_body(x1d_hbm, p_hbm, b_hbm, out_hbm, idx_v, idx_w, vals_v, vals_w,
               out_v, b_v, p_sh, p_bnc, semg0, semg1, semi0, semi1):
    cid = lax.axis_index("c")
    sid = lax.axis_index("s")
    wid = sid * NC + cid

    def stage_chunk(off, size):
        pltpu.sync_copy(p_hbm.at[pl.ds(off, size)],
                        p_bnc.at[pl.ds(0, size)])
        pltpu.sync_copy(p_bnc.at[pl.ds(0, size)],
                        p_sh.at[pl.ds(off, size)])

    stage_chunk(sid * P_SUB, P_BNC)

    @pl.when(sid < NS - 1)
    def _():
        stage_chunk(sid * P_SUB + P_BNC, P_BNC)

    @pl.when(sid == NS - 1)
    def _():
        stage_chunk((NS - 1) * P_SUB + P_BNC, P_TAIL)

    pltpu.sync_copy(b_hbm, b_v)
    bvec = b_v[...]
    base_lanes = lax.iota(jnp.int32, L) * HIST
    plsc.subcore_barrier()

    idx_bufs = (idx_v, idx_w)
    val_bufs = (vals_v, vals_w)
    gsems = (semg0, semg1)
    isems = (semi0, semi1)
    g_base = wid * GROUPS_PER_W

    def idx_src(g):
        return x1d_hbm.at[pl.ds((g_base + g) * IDX_PER_GROUP, IDX_PER_GROUP)]

    def reduce_store(g, vals):
        accs = [jnp.zeros((L,), jnp.float32) for _ in range(4)]
        for i in range(HIST):
            accs[i % 4] = accs[i % 4] + plsc.load_gather(
                vals, [base_lanes + i])
        logit = (accs[0] + accs[1]) + (accs[2] + accs[3]) + bvec
        out_v[pl.ds(g * L, L)] = _sigmoid_round(logit)

    pltpu.sync_copy(idx_src(0), idx_bufs[0])
    pltpu.async_copy(p_sh.at[idx_bufs[0]], val_bufs[0], gsems[0])
    pltpu.async_copy(idx_src(1), idx_bufs[1], isems[1])

    def pair_body(i, carry):
        for b in (0, 1):
            g = 2 * i + b
            nb = 1 - b

            @pl.when(g + 1 < GROUPS_PER_W)
            def _():
                pltpu.make_async_copy(idx_src(g + 1), idx_bufs[nb],
                                      isems[nb]).wait()
                pltpu.async_copy(p_sh.at[idx_bufs[nb]], val_bufs[nb],
                                 gsems[nb])

            pltpu.make_async_copy(p_sh.at[idx_bufs[b]], val_bufs[b],
                                  gsems[b]).wait()

            @pl.when(g + 2 < GROUPS_PER_W)
            def _():
                pltpu.async_copy(idx_src(g + 2), idx_bufs[b], isems[b])

            reduce_store(g, val_bufs[b])
        return carry

    lax.fori_loop(0, GROUPS_PER_W // 2, pair_body, 0)
    pltpu.sync_copy(out_v, out_hbm.at[pl.ds(wid * ROWS_PER_W, ROWS_PER_W)])


@functools.cache
def _build_pool_kernel():
    return pl.kernel(
        _pool_body,
        out_type=jax.ShapeDtypeStruct((BATCH,), jnp.float32),
        mesh=plsc.VectorSubcoreMesh(core_axis_name="c", subcore_axis_name="s",
                                    num_cores=NC, num_subcores=NS),
        scratch_types=[
            pltpu.VMEM((IDX_PER_GROUP,), jnp.int32),
            pltpu.VMEM((IDX_PER_GROUP,), jnp.int32),
            pltpu.VMEM((IDX_PER_GROUP,), jnp.float32),
            pltpu.VMEM((IDX_PER_GROUP,), jnp.float32),
            pltpu.VMEM((ROWS_PER_W,), jnp.float32),
            pltpu.VMEM((L,), jnp.float32),
            pltpu.VMEM_SHARED((VOCAB,), jnp.float32),
            pltpu.VMEM((P_BNC,), jnp.float32),
            pltpu.SemaphoreType.DMA,
            pltpu.SemaphoreType.DMA,
            pltpu.SemaphoreType.DMA,
            pltpu.SemaphoreType.DMA,
        ],
        compiler_params=pltpu.CompilerParams(needs_layout_passes=False),
    )


def kernel(x, table, W, b):
    w8 = jnp.broadcast_to(W.reshape(1, EMBED) * (1.0 / HIST), (8, EMBED))
    p = _project(table.T, w8)
    x1d = x.reshape(BATCH * HIST)
    b16 = jnp.broadcast_to(b.astype(jnp.float32), (L,))
    out = _build_pool_kernel()(x1d, p, b16)
    return out.reshape(BATCH, 1)
